# Initial kernel scaffold; baseline (speedup 1.0000x reference)
import jax
import jax.numpy as jnp
from jax.experimental import pallas as pl


def kernel(cells, counts_state):
    raise NotImplementedError("write your pallas kernel here")


# speedup vs baseline: 16.9938x; 16.9938x over previous
import functools

import numpy as np
import jax
import jax.numpy as jnp
from jax import lax
from jax.experimental import pallas as pl
from jax.experimental.pallas import tpu as pltpu
from jax.experimental.pallas import tpu_sc as plsc

_HASH = 16
_CELLS = 1 << _HASH
_NC, _NS, _L = 2, 16, 16
_NW = _NC * _NS
_SPR = 128 // _HASH

_W_NP = np.zeros((128, _SPR), np.float32)
for _j in range(128):
    _W_NP[_j, _j // _HASH] = float(1 << (_j % _HASH))


def _ids_body(x_ref, w_ref, o_ref):
    bits = (x_ref[...] > 0.5).astype(jnp.float32)
    ids_f = jax.lax.dot_general(
        bits, w_ref[...], (((1,), (0,)), ((), ())),
        preferred_element_type=jnp.float32)
    o_ref[...] = ids_f.astype(jnp.int32)


def _compute_ids(cells):
    n = cells.shape[0]
    rows = (n * _HASH) // 128
    blk = 1024
    x = cells.reshape(rows, 128)
    w = jnp.asarray(_W_NP)
    out = pl.pallas_call(
        _ids_body,
        grid=(rows // blk,),
        in_specs=[
            pl.BlockSpec((blk, 128), lambda i: (i, 0)),
            pl.BlockSpec((128, _SPR), lambda i: (0, 0)),
        ],
        out_specs=pl.BlockSpec((blk, _SPR), lambda i: (i, 0)),
        out_shape=jax.ShapeDtypeStruct((rows, _SPR), jnp.int32),
    )(x, w)
    return out.reshape(n // 128, 128)


def _hist_body(ids_hbm, out_hbm, idx_v, ones_v, stage_v, hist_sh):
    c = lax.axis_index("c")
    s = lax.axis_index("s")
    wid = c * _NS + s
    nrows = ids_hbm.shape[0] // _NW
    slc = _CELLS // _NS

    def _zero(i, _):
        stage_v[pl.ds(i * _L, _L)] = jnp.zeros((_L,), jnp.float32)
        return _

    lax.fori_loop(0, slc // _L, _zero, None)
    pltpu.sync_copy(stage_v, hist_sh.at[pl.ds(s * slc, slc)])

    def _one(i, _):
        ones_v[pl.ds(i * _L, _L)] = jnp.ones((_L,), jnp.float32)
        return _

    lax.fori_loop(0, 128 // _L, _one, None)

    pltpu.sync_copy(ids_hbm.at[pl.ds(wid * nrows, nrows)], idx_v)
    plsc.subcore_barrier()

    def _scat(j, _):
        pltpu.sync_copy(ones_v, hist_sh.at[idx_v.at[j]], add=True)
        return _

    lax.fori_loop(0, nrows, _scat, None)
    plsc.subcore_barrier()

    pltpu.sync_copy(hist_sh.at[pl.ds(s * slc, slc)],
                    out_hbm.at[c, pl.ds(s * slc, slc)])


_SC_PARAMS = pltpu.CompilerParams(needs_layout_passes=False)


def _hist(ids2d):
    nrows = ids2d.shape[0] // _NW
    mesh = plsc.VectorSubcoreMesh(core_axis_name="c", subcore_axis_name="s")
    return pl.kernel(
        _hist_body,
        out_type=jax.ShapeDtypeStruct((_NC, _CELLS), jnp.float32),
        mesh=mesh,
        compiler_params=_SC_PARAMS,
        scratch_types=[
            pltpu.VMEM((nrows, 128), jnp.int32),
            pltpu.VMEM((128,), jnp.float32),
            pltpu.VMEM((_CELLS // _NS,), jnp.float32),
            pltpu.VMEM_SHARED((_CELLS,), jnp.float32),
        ],
    )(ids2d)


def _tab_body(p_ref, cs_ref, o_ref):
    tot = p_ref[0] + p_ref[1] + cs_ref[...]
    o_ref[...] = jax.lax.rsqrt(jnp.maximum(tot, 1.0))


def _table(parts, counts_state):
    p = parts.reshape(_NC, _CELLS // 128, 128)
    cs = counts_state.reshape(_CELLS // 128, 128)
    out = pl.pallas_call(
        _tab_body,
        out_shape=jax.ShapeDtypeStruct((_CELLS // 128, 128), jnp.float32),
    )(p, cs)
    return out.reshape(_CELLS)


def _gather_body(ids_hbm, rtab_hbm, out_hbm, tbl_v, idx_v, res_v):
    c = lax.axis_index("c")
    s = lax.axis_index("s")
    wid = c * _NS + s
    nrows = ids_hbm.shape[0] // _NW
    half = nrows // 2

    pltpu.sync_copy(rtab_hbm, tbl_v)

    def _chunk(h, _):
        base = wid * nrows + h * half
        pltpu.sync_copy(ids_hbm.at[pl.ds(base, half)], idx_v)

        def _row(r, _r):
            def _vec(k, _k):
                vidx = idx_v[r, pl.ds(k * _L, _L)]
                res_v[r, pl.ds(k * _L, _L)] = plsc.load_gather(tbl_v, [vidx])
                return _k

            lax.fori_loop(0, 128 // _L, _vec, None)
            return _r

        lax.fori_loop(0, half, _row, None)
        pltpu.sync_copy(res_v, out_hbm.at[pl.ds(base, half)])
        return _

    lax.fori_loop(0, 2, _chunk, None)


def _gather(ids2d, rtab):
    nrows = ids2d.shape[0] // _NW
    mesh = plsc.VectorSubcoreMesh(core_axis_name="c", subcore_axis_name="s")
    return pl.kernel(
        _gather_body,
        out_type=jax.ShapeDtypeStruct(ids2d.shape, jnp.float32),
        mesh=mesh,
        compiler_params=_SC_PARAMS,
        scratch_types=[
            pltpu.VMEM((_CELLS,), jnp.float32),
            pltpu.VMEM((nrows // 2, 128), jnp.int32),
            pltpu.VMEM((nrows // 2, 128), jnp.float32),
        ],
    )(ids2d, rtab)


def kernel(cells, counts_state):
    ids2d = _compute_ids(cells)
    parts = _hist(ids2d)
    rtab = _table(parts, counts_state)
    out2d = _gather(ids2d, rtab)
    return out2d.reshape(cells.shape[0])


# speedup vs baseline: 18.4136x; 1.0835x over previous
import functools

import numpy as np
import jax
import jax.numpy as jnp
from jax import lax
from jax.experimental import pallas as pl
from jax.experimental.pallas import tpu as pltpu
from jax.experimental.pallas import tpu_sc as plsc

_HASH = 16
_CELLS = 1 << _HASH
_NC, _NS, _L = 2, 16, 16
_NW = _NC * _NS
_SPR = 128 // _HASH

def _ids_body(x_ref, o_ref):
    blk = x_ref.shape[0]
    k = lax.broadcasted_iota(jnp.int32, (1, _HASH), 1)
    powers = (1 << k).astype(jnp.float32)
    bits = (x_ref[...] > 0.5).astype(jnp.float32)
    ids_f = jnp.sum(bits * powers, axis=1)
    o_ref[...] = ids_f.astype(jnp.int32).reshape(blk // 128, 128)


def _compute_ids(cells):
    n = cells.shape[0]
    blk = 8192
    return pl.pallas_call(
        _ids_body,
        grid=(n // blk,),
        in_specs=[pl.BlockSpec((blk, _HASH), lambda i: (i, 0))],
        out_specs=pl.BlockSpec((blk // 128, 128), lambda i: (i, 0)),
        out_shape=jax.ShapeDtypeStruct((n // 128, 128), jnp.int32),
    )(cells)


def _hist_body(ids_hbm, out_hbm, idx_v, ones_v, stage_v, hist_sh):
    c = lax.axis_index("c")
    s = lax.axis_index("s")
    wid = c * _NS + s
    nrows = ids_hbm.shape[0] // _NW
    slc = _CELLS // _NS

    def _zero(i, _):
        stage_v[pl.ds(i * _L, _L)] = jnp.zeros((_L,), jnp.float32)
        return _

    lax.fori_loop(0, slc // _L, _zero, None)
    pltpu.sync_copy(stage_v, hist_sh.at[pl.ds(s * slc, slc)])

    def _one(i, _):
        ones_v[pl.ds(i * _L, _L)] = jnp.ones((_L,), jnp.float32)
        return _

    lax.fori_loop(0, 128 // _L, _one, None)

    pltpu.sync_copy(ids_hbm.at[pl.ds(wid * nrows, nrows)], idx_v)
    plsc.subcore_barrier()

    def _scat(j, _):
        pltpu.sync_copy(ones_v, hist_sh.at[idx_v.at[j]], add=True)
        return _

    lax.fori_loop(0, nrows, _scat, None)
    plsc.subcore_barrier()

    pltpu.sync_copy(hist_sh.at[pl.ds(s * slc, slc)],
                    out_hbm.at[c, pl.ds(s * slc, slc)])


_SC_PARAMS = pltpu.CompilerParams(needs_layout_passes=False)


def _hist(ids2d):
    nrows = ids2d.shape[0] // _NW
    mesh = plsc.VectorSubcoreMesh(core_axis_name="c", subcore_axis_name="s")
    return pl.kernel(
        _hist_body,
        out_type=jax.ShapeDtypeStruct((_NC, _CELLS), jnp.float32),
        mesh=mesh,
        compiler_params=_SC_PARAMS,
        scratch_types=[
            pltpu.VMEM((nrows, 128), jnp.int32),
            pltpu.VMEM((128,), jnp.float32),
            pltpu.VMEM((_CELLS // _NS,), jnp.float32),
            pltpu.VMEM_SHARED((_CELLS,), jnp.float32),
        ],
    )(ids2d)


def _tab_body(p_ref, cs_ref, o_ref):
    tot = p_ref[0] + p_ref[1] + cs_ref[...]
    o_ref[...] = jax.lax.rsqrt(jnp.maximum(tot, 1.0))


def _table(parts, counts_state):
    return pl.pallas_call(
        _tab_body,
        out_shape=jax.ShapeDtypeStruct((_CELLS,), jnp.float32),
    )(parts, counts_state)


def _gather_body(ids_hbm, rtab_hbm, out_hbm, tbl_v, idx_v, res_v):
    c = lax.axis_index("c")
    s = lax.axis_index("s")
    wid = c * _NS + s
    nrows = ids_hbm.shape[0] // _NW
    half = nrows // 2

    pltpu.sync_copy(rtab_hbm, tbl_v)

    def _chunk(h, _):
        base = wid * nrows + h * half
        pltpu.sync_copy(ids_hbm.at[pl.ds(base, half)], idx_v)

        def _row(r, _r):
            def _vec(k, _k):
                vidx = idx_v[r, pl.ds(k * _L, _L)]
                res_v[r, pl.ds(k * _L, _L)] = plsc.load_gather(tbl_v, [vidx])
                return _k

            lax.fori_loop(0, 128 // _L, _vec, None)
            return _r

        lax.fori_loop(0, half, _row, None)
        pltpu.sync_copy(res_v, out_hbm.at[pl.ds(base, half)])
        return _

    lax.fori_loop(0, 2, _chunk, None)


def _gather(ids2d, rtab):
    nrows = ids2d.shape[0] // _NW
    mesh = plsc.VectorSubcoreMesh(core_axis_name="c", subcore_axis_name="s")
    return pl.kernel(
        _gather_body,
        out_type=jax.ShapeDtypeStruct(ids2d.shape, jnp.float32),
        mesh=mesh,
        compiler_params=_SC_PARAMS,
        scratch_types=[
            pltpu.VMEM((_CELLS,), jnp.float32),
            pltpu.VMEM((nrows // 2, 128), jnp.int32),
            pltpu.VMEM((nrows // 2, 128), jnp.float32),
        ],
    )(ids2d, rtab)


def kernel(cells, counts_state):
    ids2d = _compute_ids(cells)
    parts = _hist(ids2d)
    rtab = _table(parts, counts_state)
    out2d = _gather(ids2d, rtab)
    return out2d.reshape(cells.shape[0])


# speedup vs baseline: 94.4606x; 5.1299x over previous
import functools

import numpy as np
import jax
import jax.numpy as jnp
from jax import lax
from jax.experimental import pallas as pl
from jax.experimental.pallas import tpu as pltpu
from jax.experimental.pallas import tpu_sc as plsc

_HASH = 16
_CELLS = 1 << _HASH
_NC, _NS, _L = 2, 16, 16
_NW = _NC * _NS
_SPR = 128 // _HASH

def _ids_body(xt_ref, o_ref):
    blk = xt_ref.shape[1]
    k = lax.broadcasted_iota(jnp.int32, (_HASH, 1), 0)
    powers = (1 << k).astype(jnp.float32)
    bits = (xt_ref[...] > 0.5).astype(jnp.float32)
    ids_f = jnp.sum(bits * powers, axis=0)
    o_ref[...] = ids_f.astype(jnp.int32).reshape(blk // 128, 128)


def _compute_ids(cells):
    n = cells.shape[0]
    blk = 32768
    return pl.pallas_call(
        _ids_body,
        grid=(n // blk,),
        in_specs=[pl.BlockSpec((_HASH, blk), lambda i: (0, i))],
        out_specs=pl.BlockSpec((blk // 128, 128), lambda i: (i, 0)),
        out_shape=jax.ShapeDtypeStruct((n // 128, 128), jnp.int32),
    )(cells.T)


def _hist_body(ids_hbm, out_hbm, idx_v, ones_v, stage_v, hist_sh):
    c = lax.axis_index("c")
    s = lax.axis_index("s")
    wid = c * _NS + s
    nrows = ids_hbm.shape[0] // _NW
    slc = _CELLS // _NS

    def _zero(i, _):
        stage_v[pl.ds(i * _L, _L)] = jnp.zeros((_L,), jnp.float32)
        return _

    lax.fori_loop(0, slc // _L, _zero, None)
    pltpu.sync_copy(stage_v, hist_sh.at[pl.ds(s * slc, slc)])

    def _one(i, _):
        ones_v[pl.ds(i * _L, _L)] = jnp.ones((_L,), jnp.float32)
        return _

    lax.fori_loop(0, 128 // _L, _one, None)

    pltpu.sync_copy(ids_hbm.at[pl.ds(wid * nrows, nrows)], idx_v)
    plsc.subcore_barrier()

    def _scat(j, _):
        pltpu.sync_copy(ones_v, hist_sh.at[idx_v.at[j]], add=True)
        return _

    lax.fori_loop(0, nrows, _scat, None)
    plsc.subcore_barrier()

    pltpu.sync_copy(hist_sh.at[pl.ds(s * slc, slc)],
                    out_hbm.at[c, pl.ds(s * slc, slc)])


_SC_PARAMS = pltpu.CompilerParams(needs_layout_passes=False)


def _hist(ids2d):
    nrows = ids2d.shape[0] // _NW
    mesh = plsc.VectorSubcoreMesh(core_axis_name="c", subcore_axis_name="s")
    return pl.kernel(
        _hist_body,
        out_type=jax.ShapeDtypeStruct((_NC, _CELLS), jnp.float32),
        mesh=mesh,
        compiler_params=_SC_PARAMS,
        scratch_types=[
            pltpu.VMEM((nrows, 128), jnp.int32),
            pltpu.VMEM((128,), jnp.float32),
            pltpu.VMEM((_CELLS // _NS,), jnp.float32),
            pltpu.VMEM_SHARED((_CELLS,), jnp.float32),
        ],
    )(ids2d)


def _tab_body(p_ref, cs_ref, o_ref):
    tot = p_ref[0] + p_ref[1] + cs_ref[...]
    o_ref[...] = jax.lax.rsqrt(jnp.maximum(tot, 1.0))


def _table(parts, counts_state):
    return pl.pallas_call(
        _tab_body,
        out_shape=jax.ShapeDtypeStruct((_CELLS,), jnp.float32),
    )(parts, counts_state)


def _gather_body(ids_hbm, rtab_hbm, out_hbm, tbl_v, idx_v, res_v):
    c = lax.axis_index("c")
    s = lax.axis_index("s")
    wid = c * _NS + s
    nrows = ids_hbm.shape[0] // _NW
    half = nrows // 2

    pltpu.sync_copy(rtab_hbm, tbl_v)

    def _chunk(h, _):
        base = wid * nrows + h * half
        pltpu.sync_copy(ids_hbm.at[pl.ds(base, half)], idx_v)

        def _row(r, _r):
            def _vec(k, _k):
                vidx = idx_v[r, pl.ds(k * _L, _L)]
                res_v[r, pl.ds(k * _L, _L)] = plsc.load_gather(tbl_v, [vidx])
                return _k

            lax.fori_loop(0, 128 // _L, _vec, None)
            return _r

        lax.fori_loop(0, half, _row, None)
        pltpu.sync_copy(res_v, out_hbm.at[pl.ds(base, half)])
        return _

    lax.fori_loop(0, 2, _chunk, None)


def _gather(ids2d, rtab):
    nrows = ids2d.shape[0] // _NW
    mesh = plsc.VectorSubcoreMesh(core_axis_name="c", subcore_axis_name="s")
    return pl.kernel(
        _gather_body,
        out_type=jax.ShapeDtypeStruct(ids2d.shape, jnp.float32),
        mesh=mesh,
        compiler_params=_SC_PARAMS,
        scratch_types=[
            pltpu.VMEM((_CELLS,), jnp.float32),
            pltpu.VMEM((nrows // 2, 128), jnp.int32),
            pltpu.VMEM((nrows // 2, 128), jnp.float32),
        ],
    )(ids2d, rtab)


def kernel(cells, counts_state):
    ids2d = _compute_ids(cells)
    parts = _hist(ids2d)
    rtab = _table(parts, counts_state)
    out2d = _gather(ids2d, rtab)
    return out2d.reshape(cells.shape[0])


# speedup vs baseline: 117.5165x; 1.2441x over previous
import functools

import numpy as np
import jax
import jax.numpy as jnp
from jax import lax
from jax.experimental import pallas as pl
from jax.experimental.pallas import tpu as pltpu
from jax.experimental.pallas import tpu_sc as plsc

_HASH = 16
_CELLS = 1 << _HASH
_NC, _NS, _L = 2, 16, 16
_NW = _NC * _NS
_SPR = 128 // _HASH

def _ids_body(xt_ref, o_ref):
    blk = xt_ref.shape[1]
    k = lax.broadcasted_iota(jnp.int32, (_HASH, 1), 0)
    powers = (1 << k).astype(jnp.float32)
    bits = (xt_ref[...] > 0.5).astype(jnp.float32)
    ids_f = jnp.sum(bits * powers, axis=0)
    o_ref[...] = ids_f.astype(jnp.int32).reshape(blk // 128, 128)


def _compute_ids(cells):
    n = cells.shape[0]
    blk = 32768
    return pl.pallas_call(
        _ids_body,
        grid=(n // blk,),
        in_specs=[pl.BlockSpec((_HASH, blk), lambda i: (0, i))],
        out_specs=pl.BlockSpec((blk // 128, 128), lambda i: (i, 0)),
        out_shape=jax.ShapeDtypeStruct((n // 128, 128), jnp.int32),
    )(cells.T)


def _hist_body(ids_hbm, out_hbm, idx_v, ones_v, stage_v, hist_sh, sem):
    c = lax.axis_index("c")
    s = lax.axis_index("s")
    wid = c * _NS + s
    nrows = ids_hbm.shape[0] // _NW
    slc = _CELLS // _NS

    @plsc.parallel_loop(0, slc // _L, 1, unroll=8)
    def _zero(i):
        stage_v[pl.ds(i * _L, _L)] = jnp.zeros((_L,), jnp.float32)

    pltpu.sync_copy(stage_v, hist_sh.at[pl.ds(s * slc, slc)])

    @plsc.parallel_loop(0, 128 // _L, 1, unroll=8)
    def _one(i):
        ones_v[pl.ds(i * _L, _L)] = jnp.ones((_L,), jnp.float32)

    pltpu.sync_copy(ids_hbm.at[pl.ds(wid * nrows, nrows)], idx_v)
    plsc.subcore_barrier()

    k = 8

    def _grp(g, _):
        descs = []
        for u in range(k):
            descs.append(pltpu.async_copy(
                ones_v, hist_sh.at[idx_v.at[g * k + u]], sem, add=True))
        for d in descs:
            d.wait()
        return _

    lax.fori_loop(0, nrows // k, _grp, None)
    plsc.subcore_barrier()

    pltpu.sync_copy(hist_sh.at[pl.ds(s * slc, slc)],
                    out_hbm.at[c, pl.ds(s * slc, slc)])


_SC_PARAMS = pltpu.CompilerParams(needs_layout_passes=False)


def _hist(ids2d):
    nrows = ids2d.shape[0] // _NW
    mesh = plsc.VectorSubcoreMesh(core_axis_name="c", subcore_axis_name="s")
    return pl.kernel(
        _hist_body,
        out_type=jax.ShapeDtypeStruct((_NC, _CELLS), jnp.float32),
        mesh=mesh,
        compiler_params=_SC_PARAMS,
        scratch_types=[
            pltpu.VMEM((nrows, 128), jnp.int32),
            pltpu.VMEM((128,), jnp.float32),
            pltpu.VMEM((_CELLS // _NS,), jnp.float32),
            pltpu.VMEM_SHARED((_CELLS,), jnp.float32),
            pltpu.SemaphoreType.DMA,
        ],
    )(ids2d)


def _tab_body(p_ref, cs_ref, o_ref):
    tot = p_ref[0] + p_ref[1] + cs_ref[...]
    o_ref[...] = jax.lax.rsqrt(jnp.maximum(tot, 1.0))


def _table(parts, counts_state):
    return pl.pallas_call(
        _tab_body,
        out_shape=jax.ShapeDtypeStruct((_CELLS,), jnp.float32),
    )(parts, counts_state)


def _gather_body(ids_hbm, rtab_hbm, out_hbm, tbl_v, idx_v, res_v):
    c = lax.axis_index("c")
    s = lax.axis_index("s")
    wid = c * _NS + s
    nrows = ids_hbm.shape[0] // _NW
    half = nrows // 2

    pltpu.sync_copy(rtab_hbm, tbl_v)

    def _chunk(h, _):
        base = wid * nrows + h * half
        pltpu.sync_copy(ids_hbm.at[pl.ds(base, half)], idx_v)

        @plsc.parallel_loop(0, half * (128 // _L), 1, unroll=8)
        def _vec(i):
            r = i >> 3
            k = i & 7
            vidx = idx_v[r, pl.ds(k * _L, _L)]
            res_v[r, pl.ds(k * _L, _L)] = plsc.load_gather(tbl_v, [vidx])

        pltpu.sync_copy(res_v, out_hbm.at[pl.ds(base, half)])
        return _

    lax.fori_loop(0, 2, _chunk, None)


def _gather(ids2d, rtab):
    nrows = ids2d.shape[0] // _NW
    mesh = plsc.VectorSubcoreMesh(core_axis_name="c", subcore_axis_name="s")
    return pl.kernel(
        _gather_body,
        out_type=jax.ShapeDtypeStruct(ids2d.shape, jnp.float32),
        mesh=mesh,
        compiler_params=_SC_PARAMS,
        scratch_types=[
            pltpu.VMEM((_CELLS,), jnp.float32),
            pltpu.VMEM((nrows // 2, 128), jnp.int32),
            pltpu.VMEM((nrows // 2, 128), jnp.float32),
        ],
    )(ids2d, rtab)


def kernel(cells, counts_state):
    ids2d = _compute_ids(cells)
    parts = _hist(ids2d)
    rtab = _table(parts, counts_state)
    out2d = _gather(ids2d, rtab)
    return out2d.reshape(cells.shape[0])


# speedup vs baseline: 132.2773x; 1.1256x over previous
import functools

import numpy as np
import jax
import jax.numpy as jnp
from jax import lax
from jax.experimental import pallas as pl
from jax.experimental.pallas import tpu as pltpu
from jax.experimental.pallas import tpu_sc as plsc

_HASH = 16
_CELLS = 1 << _HASH
_NC, _NS, _L = 2, 16, 16
_NW = _NC * _NS
_SPR = 128 // _HASH

def _ids_body(xt_ref, o_ref):
    blk = xt_ref.shape[1]
    k = lax.broadcasted_iota(jnp.int32, (_HASH, 1), 0)
    raw = jax.lax.bitcast_convert_type(xt_ref[...], jnp.int32)
    bits = jax.lax.shift_right_logical(raw, 29) & 1
    ids = jnp.sum(bits << k, axis=0)
    o_ref[...] = ids.reshape(blk // 128, 128)


def _compute_ids(cells):
    n = cells.shape[0]
    blk = 65536
    return pl.pallas_call(
        _ids_body,
        grid=(n // blk,),
        in_specs=[pl.BlockSpec((_HASH, blk), lambda i: (0, i))],
        out_specs=pl.BlockSpec((blk // 128, 128), lambda i: (i, 0)),
        out_shape=jax.ShapeDtypeStruct((n // 128, 128), jnp.int32),
    )(cells.T)


def _hist_body(ids_hbm, out_hbm, idx_v, ones_v, stage_v, hist_sh, sem):
    c = lax.axis_index("c")
    s = lax.axis_index("s")
    wid = c * _NS + s
    nrows = ids_hbm.shape[0] // _NW
    slc = _CELLS // _NS

    @plsc.parallel_loop(0, slc // _L, 1, unroll=8)
    def _zero(i):
        stage_v[pl.ds(i * _L, _L)] = jnp.zeros((_L,), jnp.float32)

    pltpu.sync_copy(stage_v, hist_sh.at[pl.ds(s * slc, slc)])

    @plsc.parallel_loop(0, 128 // _L, 1, unroll=8)
    def _one(i):
        ones_v[pl.ds(i * _L, _L)] = jnp.ones((_L,), jnp.float32)

    pltpu.sync_copy(ids_hbm.at[pl.ds(wid * nrows, nrows)], idx_v)
    plsc.subcore_barrier()

    k = 16

    def _grp(g, _):
        descs = []
        for u in range(k):
            descs.append(pltpu.async_copy(
                ones_v, hist_sh.at[idx_v.at[g * k + u]], sem, add=True))
        for d in descs:
            d.wait()
        return _

    lax.fori_loop(0, nrows // k, _grp, None)
    plsc.subcore_barrier()

    pltpu.sync_copy(hist_sh.at[pl.ds(s * slc, slc)],
                    out_hbm.at[c, pl.ds(s * slc, slc)])


_SC_PARAMS = pltpu.CompilerParams(needs_layout_passes=False)


def _hist(ids2d):
    nrows = ids2d.shape[0] // _NW
    mesh = plsc.VectorSubcoreMesh(core_axis_name="c", subcore_axis_name="s")
    return pl.kernel(
        _hist_body,
        out_type=jax.ShapeDtypeStruct((_NC, _CELLS), jnp.float32),
        mesh=mesh,
        compiler_params=_SC_PARAMS,
        scratch_types=[
            pltpu.VMEM((nrows, 128), jnp.int32),
            pltpu.VMEM((128,), jnp.float32),
            pltpu.VMEM((_CELLS // _NS,), jnp.float32),
            pltpu.VMEM_SHARED((_CELLS,), jnp.float32),
            pltpu.SemaphoreType.DMA,
        ],
    )(ids2d)


def _tab_body(p_ref, cs_ref, o_ref):
    tot = p_ref[0] + p_ref[1] + cs_ref[...]
    o_ref[...] = jax.lax.rsqrt(jnp.maximum(tot, 1.0))


def _table(parts, counts_state):
    return pl.pallas_call(
        _tab_body,
        out_shape=jax.ShapeDtypeStruct((_CELLS,), jnp.float32),
    )(parts, counts_state)


def _gather_body(ids_hbm, rtab_hbm, out_hbm, tbl_v, idx_v, res_v):
    c = lax.axis_index("c")
    s = lax.axis_index("s")
    wid = c * _NS + s
    nrows = ids_hbm.shape[0] // _NW
    half = nrows // 2

    pltpu.sync_copy(rtab_hbm, tbl_v)

    def _chunk(h, _):
        base = wid * nrows + h * half
        pltpu.sync_copy(ids_hbm.at[pl.ds(base, half)], idx_v)

        @plsc.parallel_loop(0, half * (128 // _L), 1, unroll=16)
        def _vec(i):
            r = i >> 3
            k = i & 7
            vidx = idx_v[r, pl.ds(k * _L, _L)]
            res_v[r, pl.ds(k * _L, _L)] = plsc.load_gather(tbl_v, [vidx])

        pltpu.sync_copy(res_v, out_hbm.at[pl.ds(base, half)])
        return _

    lax.fori_loop(0, 2, _chunk, None)


def _gather(ids2d, rtab):
    nrows = ids2d.shape[0] // _NW
    mesh = plsc.VectorSubcoreMesh(core_axis_name="c", subcore_axis_name="s")
    return pl.kernel(
        _gather_body,
        out_type=jax.ShapeDtypeStruct(ids2d.shape, jnp.float32),
        mesh=mesh,
        compiler_params=_SC_PARAMS,
        scratch_types=[
            pltpu.VMEM((_CELLS,), jnp.float32),
            pltpu.VMEM((nrows // 2, 128), jnp.int32),
            pltpu.VMEM((nrows // 2, 128), jnp.float32),
        ],
    )(ids2d, rtab)


def kernel(cells, counts_state):
    ids2d = _compute_ids(cells)
    parts = _hist(ids2d)
    rtab = _table(parts, counts_state)
    out2d = _gather(ids2d, rtab)
    return out2d.reshape(cells.shape[0])


# speedup vs baseline: 137.7100x; 1.0411x over previous
import functools

import numpy as np
import jax
import jax.numpy as jnp
from jax import lax
from jax.experimental import pallas as pl
from jax.experimental.pallas import tpu as pltpu
from jax.experimental.pallas import tpu_sc as plsc

_HASH = 16
_CELLS = 1 << _HASH
_NC, _NS, _L = 2, 16, 16
_NW = _NC * _NS
_SPR = 128 // _HASH

def _ids_body(xt_ref, o_ref):
    blk = xt_ref.shape[1]
    k = lax.broadcasted_iota(jnp.int32, (_HASH, 1), 0)
    raw = jax.lax.bitcast_convert_type(xt_ref[...], jnp.int32)
    bits = jax.lax.shift_right_logical(raw, 29) & 1
    ids = jnp.sum(bits << k, axis=0)
    o_ref[...] = ids.reshape(blk // 128, 128)


def _compute_ids(xt, start_blk, nblk):
    blk = 65536
    return pl.pallas_call(
        _ids_body,
        grid=(nblk,),
        in_specs=[pl.BlockSpec((_HASH, blk), lambda i: (0, i + start_blk))],
        out_specs=pl.BlockSpec((blk // 128, 128), lambda i: (i, 0)),
        out_shape=jax.ShapeDtypeStruct((nblk * blk // 128, 128), jnp.int32),
    )(xt)


def _hist_body(ids_hbm, out_hbm, idx_v, ones_v, stage_v, hist_sh, sem):
    c = lax.axis_index("c")
    s = lax.axis_index("s")
    wid = c * _NS + s
    nrows = ids_hbm.shape[0] // _NW
    slc = _CELLS // _NS

    @plsc.parallel_loop(0, slc // _L, 1, unroll=8)
    def _zero(i):
        stage_v[pl.ds(i * _L, _L)] = jnp.zeros((_L,), jnp.float32)

    pltpu.sync_copy(stage_v, hist_sh.at[pl.ds(s * slc, slc)])

    @plsc.parallel_loop(0, 128 // _L, 1, unroll=8)
    def _one(i):
        ones_v[pl.ds(i * _L, _L)] = jnp.ones((_L,), jnp.float32)

    pltpu.sync_copy(ids_hbm.at[pl.ds(wid * nrows, nrows)], idx_v)
    plsc.subcore_barrier()

    k = 16

    def _grp(g, _):
        descs = []
        for u in range(k):
            descs.append(pltpu.async_copy(
                ones_v, hist_sh.at[idx_v.at[g * k + u]], sem, add=True))
        for d in descs:
            d.wait()
        return _

    lax.fori_loop(0, nrows // k, _grp, None)
    plsc.subcore_barrier()

    pltpu.sync_copy(hist_sh.at[pl.ds(s * slc, slc)],
                    out_hbm.at[c, pl.ds(s * slc, slc)])


_SC_PARAMS = pltpu.CompilerParams(needs_layout_passes=False)


def _hist(ids2d):
    nrows = ids2d.shape[0] // _NW
    mesh = plsc.VectorSubcoreMesh(core_axis_name="c", subcore_axis_name="s")
    return pl.kernel(
        _hist_body,
        out_type=jax.ShapeDtypeStruct((_NC, _CELLS), jnp.float32),
        mesh=mesh,
        compiler_params=_SC_PARAMS,
        scratch_types=[
            pltpu.VMEM((nrows, 128), jnp.int32),
            pltpu.VMEM((128,), jnp.float32),
            pltpu.VMEM((_CELLS // _NS,), jnp.float32),
            pltpu.VMEM_SHARED((_CELLS,), jnp.float32),
            pltpu.SemaphoreType.DMA,
        ],
    )(ids2d)


def _tab_body(pa_ref, pb_ref, cs_ref, o_ref):
    tot = (pa_ref[0] + pa_ref[1]) + (pb_ref[0] + pb_ref[1]) + cs_ref[...]
    o_ref[...] = jax.lax.rsqrt(jnp.maximum(tot, 1.0))


def _table(parts_a, parts_b, counts_state):
    return pl.pallas_call(
        _tab_body,
        out_shape=jax.ShapeDtypeStruct((_CELLS,), jnp.float32),
    )(parts_a, parts_b, counts_state)


def _gather_body(ids_a, ids_b, rtab_hbm, out_hbm, tbl_v, idx_v, res_v):
    c = lax.axis_index("c")
    s = lax.axis_index("s")
    wid = c * _NS + s
    half = ids_a.shape[0] // _NW

    pltpu.sync_copy(rtab_hbm, tbl_v)

    for h, ids_h in enumerate((ids_a, ids_b)):
        base = wid * half
        pltpu.sync_copy(ids_h.at[pl.ds(base, half)], idx_v)

        @plsc.parallel_loop(0, half * (128 // _L), 1, unroll=16)
        def _vec(i):
            r = i >> 3
            k = i & 7
            vidx = idx_v[r, pl.ds(k * _L, _L)]
            res_v[r, pl.ds(k * _L, _L)] = plsc.load_gather(tbl_v, [vidx])

        pltpu.sync_copy(
            res_v, out_hbm.at[pl.ds(h * ids_a.shape[0] + base, half)])


def _gather(ids_a, ids_b, rtab):
    half = ids_a.shape[0] // _NW
    mesh = plsc.VectorSubcoreMesh(core_axis_name="c", subcore_axis_name="s")
    return pl.kernel(
        _gather_body,
        out_type=jax.ShapeDtypeStruct((2 * ids_a.shape[0], 128), jnp.float32),
        mesh=mesh,
        compiler_params=_SC_PARAMS,
        scratch_types=[
            pltpu.VMEM((_CELLS,), jnp.float32),
            pltpu.VMEM((half, 128), jnp.int32),
            pltpu.VMEM((half, 128), jnp.float32),
        ],
    )(ids_a, ids_b, rtab)


def kernel(cells, counts_state):
    n = cells.shape[0]
    xt = cells.T
    nblk = (n // 65536) // 2
    ids_a = _compute_ids(xt, 0, nblk)
    parts_a = _hist(ids_a)
    ids_b = _compute_ids(xt, nblk, nblk)
    parts_b = _hist(ids_b)
    rtab = _table(parts_a, parts_b, counts_state)
    out2d = _gather(ids_a, ids_b, rtab)
    return out2d.reshape(n)
